# baseline (speedup 1.0000x reference)
import functools

import jax
import jax.numpy as jnp
from jax import lax
from jax.experimental import pallas as pl
from jax.experimental.pallas import tpu as pltpu
from jax.experimental.pallas import tpu_sc as plsc

_N = 10000
_E = 160000
_D = 256
_H = _D // 2
_NC = 2
_NS = 16
_EPT = _E // _NS
_C = 128
_NCHUNK = 80
_EPTP = _NCHUNK * _C
_NACC = _N + 16
_RPT = 624
_RTAIL = _N - _NS * _RPT
_BN = 1000


def _seg_sum_sc(yA, yB, src, dst, zrow, zdeg, with_deg):
    mesh = plsc.VectorSubcoreMesh(core_axis_name="c", subcore_axis_name="s")
    out_type = [jax.ShapeDtypeStruct((_NC, _N, _H), jnp.float32)]
    if with_deg:
        out_type.append(jax.ShapeDtypeStruct((_NACC,), jnp.float32))

    @functools.partial(
        pl.kernel,
        out_type=tuple(out_type),
        mesh=mesh,
        scratch_types=[
            pltpu.VMEM((_EPTP,), jnp.int32),
            pltpu.VMEM((_C,), jnp.int32),
            pltpu.VMEM((_C,), jnp.int32),
            pltpu.VMEM((_C, _H), jnp.float32),
            pltpu.VMEM((_C, _H), jnp.float32),
            pltpu.VMEM((_C,), jnp.float32),
            pltpu.VMEM_SHARED((_NACC, _H), jnp.float32),
            pltpu.VMEM_SHARED((_NACC,), jnp.float32),
            pltpu.SemaphoreType.DMA,
            pltpu.SemaphoreType.DMA,
            pltpu.SemaphoreType.DMA,
            pltpu.SemaphoreType.DMA,
        ],
    )
    def k(yA_hbm, yB_hbm, src_hbm, dst_hbm, zrow_hbm, zdeg_hbm,
          *out_and_scratch):
        if with_deg:
            s_out, deg_out = out_and_scratch[:2]
            scr = out_and_scratch[2:]
        else:
            s_out = out_and_scratch[0]
            scr = out_and_scratch[1:]
        (src_st, dstA, dstB, bufA, bufB, ones_v,
         acc_sh, deg_sh, semA, semB, dsemA, dsemB) = scr
        c = lax.axis_index("c")
        s = lax.axis_index("s")
        if with_deg:
            for i in range(_C // 16):
                ones_v[pl.ds(i * 16, 16)] = jnp.ones((16,), jnp.float32)

        base = s * _EPTP
        pltpu.async_copy(src_hbm.at[pl.ds(base, _EPTP)], src_st, semA)

        @pl.when(s == 0)
        def _():
            pltpu.sync_copy(zrow_hbm, acc_sh)
            if with_deg:
                @pl.when(c == 0)
                def _():
                    pltpu.sync_copy(zdeg_hbm, deg_sh)

        pltpu.make_async_copy(src_hbm.at[pl.ds(0, _EPTP)], src_st, semA).wait()
        plsc.subcore_barrier()

        def gather(kk, buf, sem):
            idx = src_st.at[pl.ds(kk * _C, _C)]

            @pl.when(c == 0)
            def _():
                pltpu.async_copy(yA_hbm.at[idx], buf, sem)

            @pl.when(c == 1)
            def _():
                pltpu.async_copy(yB_hbm.at[idx], buf, sem)

        def wait(buf, sem):
            pltpu.make_async_copy(
                yA_hbm.at[src_st.at[pl.ds(0, _C)]], buf, sem).wait()

        def fetch_dst(kk, dst_v, dsem):
            pltpu.async_copy(dst_hbm.at[pl.ds(base + kk * _C, _C)],
                             dst_v, dsem)

        def wait_dst(dst_v, dsem):
            pltpu.make_async_copy(
                dst_hbm.at[pl.ds(0, _C)], dst_v, dsem).wait()

        def scatter(kk, buf, dst_v):
            pltpu.sync_copy(buf, acc_sh.at[dst_v], add=True)
            if with_deg:
                @pl.when(c == 0)
                def _():
                    pltpu.sync_copy(ones_v, deg_sh.at[dst_v], add=True)

        gather(0, bufA, semA)
        fetch_dst(0, dstA, dsemA)

        @pl.loop(0, (_NCHUNK - 2) // 2)
        def _(i):
            k0 = 2 * i
            gather(k0 + 1, bufB, semB)
            fetch_dst(k0 + 1, dstB, dsemB)
            wait(bufA, semA)
            wait_dst(dstA, dsemA)
            scatter(k0, bufA, dstA)
            gather(k0 + 2, bufA, semA)
            fetch_dst(k0 + 2, dstA, dsemA)
            wait(bufB, semB)
            wait_dst(dstB, dsemB)
            scatter(k0 + 1, bufB, dstB)

        gather(_NCHUNK - 1, bufB, semB)
        fetch_dst(_NCHUNK - 1, dstB, dsemB)
        wait(bufA, semA)
        wait_dst(dstA, dsemA)
        scatter(_NCHUNK - 2, bufA, dstA)
        wait(bufB, semB)
        wait_dst(dstB, dsemB)
        scatter(_NCHUNK - 1, bufB, dstB)

        plsc.subcore_barrier()
        pltpu.sync_copy(acc_sh.at[pl.ds(s * _RPT, _RPT)],
                        s_out.at[c, pl.ds(s * _RPT, _RPT)])

        @pl.when(s == _NS - 1)
        def _():
            pltpu.sync_copy(acc_sh.at[pl.ds(_NS * _RPT, _RTAIL)],
                            s_out.at[c, pl.ds(_NS * _RPT, _RTAIL)])

        if with_deg:
            @pl.when(jnp.logical_and(s == 0, c == 0))
            def _():
                pltpu.sync_copy(deg_sh, deg_out)

    return k(yA, yB, src, dst, zrow, zdeg)


def _mm(a, b):
    return jnp.dot(a, b, preferred_element_type=jnp.float32)


def _tc_pre(x, Wl, Wr, b):
    def body(x_ref, wl_ref, wr_ref, b_ref, yA_ref, yB_ref, z_ref):
        xb = x_ref[...]
        y = _mm(xb, wl_ref[...])
        yA_ref[...] = y[:, :_H]
        yB_ref[...] = y[:, _H:]
        z_ref[...] = _mm(xb, wr_ref[...]) + b_ref[...]

    return pl.pallas_call(
        body,
        grid=(_N // _BN,),
        in_specs=[
            pl.BlockSpec((_BN, _D), lambda i: (i, 0)),
            pl.BlockSpec((_D, _D), lambda i: (0, 0)),
            pl.BlockSpec((_D, _D), lambda i: (0, 0)),
            pl.BlockSpec((1, _D), lambda i: (0, 0)),
        ],
        out_specs=[
            pl.BlockSpec((_BN, _H), lambda i: (i, 0)),
            pl.BlockSpec((_BN, _H), lambda i: (i, 0)),
            pl.BlockSpec((_BN, _D), lambda i: (i, 0)),
        ],
        out_shape=[
            jax.ShapeDtypeStruct((_N, _H), jnp.float32),
            jax.ShapeDtypeStruct((_N, _H), jnp.float32),
            jax.ShapeDtypeStruct((_N, _D), jnp.float32),
        ],
    )(x, Wl, Wr, b.reshape(1, _D))


def _tc_mid(sA, sB, deg, z, Wl, Wr, b):
    def body(sA_ref, sB_ref, deg_ref, z_ref, wl_ref, wr_ref, b_ref,
             yA_ref, yB_ref, z2_ref):
        rd = 1.0 / jnp.maximum(deg_ref[...], 1.0)
        zb = z_ref[...]
        hA = jnp.maximum(sA_ref[...] * rd + zb[:, :_H], 0.0)
        hB = jnp.maximum(sB_ref[...] * rd + zb[:, _H:], 0.0)
        wl = wl_ref[...]
        wr = wr_ref[...]
        y2 = _mm(hA, wl[:_H, :]) + _mm(hB, wl[_H:, :])
        yA_ref[...] = y2[:, :_H]
        yB_ref[...] = y2[:, _H:]
        z2_ref[...] = _mm(hA, wr[:_H, :]) + _mm(hB, wr[_H:, :]) + b_ref[...]

    return pl.pallas_call(
        body,
        grid=(_N // _BN,),
        in_specs=[
            pl.BlockSpec((_BN, _H), lambda i: (i, 0)),
            pl.BlockSpec((_BN, _H), lambda i: (i, 0)),
            pl.BlockSpec((_BN, 1), lambda i: (i, 0)),
            pl.BlockSpec((_BN, _D), lambda i: (i, 0)),
            pl.BlockSpec((_D, _D), lambda i: (0, 0)),
            pl.BlockSpec((_D, _D), lambda i: (0, 0)),
            pl.BlockSpec((1, _D), lambda i: (0, 0)),
        ],
        out_specs=[
            pl.BlockSpec((_BN, _H), lambda i: (i, 0)),
            pl.BlockSpec((_BN, _H), lambda i: (i, 0)),
            pl.BlockSpec((_BN, _D), lambda i: (i, 0)),
        ],
        out_shape=[
            jax.ShapeDtypeStruct((_N, _H), jnp.float32),
            jax.ShapeDtypeStruct((_N, _H), jnp.float32),
            jax.ShapeDtypeStruct((_N, _D), jnp.float32),
        ],
    )(sA, sB, deg, z, Wl, Wr, b.reshape(1, _D))


def _tc_post(sA, sB, deg, z):
    def body(sA_ref, sB_ref, deg_ref, z_ref, o_ref):
        rd = 1.0 / jnp.maximum(deg_ref[...], 1.0)
        o_ref[...] = jnp.concatenate(
            [sA_ref[...] * rd, sB_ref[...] * rd], axis=1) + z_ref[...]

    return pl.pallas_call(
        body,
        grid=(_N // _BN,),
        in_specs=[
            pl.BlockSpec((_BN, _H), lambda i: (i, 0)),
            pl.BlockSpec((_BN, _H), lambda i: (i, 0)),
            pl.BlockSpec((_BN, 1), lambda i: (i, 0)),
            pl.BlockSpec((_BN, _D), lambda i: (i, 0)),
        ],
        out_specs=pl.BlockSpec((_BN, _D), lambda i: (i, 0)),
        out_shape=jax.ShapeDtypeStruct((_N, _D), jnp.float32),
    )(sA, sB, deg, z)


def kernel(x, edge_index, Wl1, Wr1, b1, Wl2, Wr2, b2):
    pad = _EPTP - _EPT
    src = edge_index[0].astype(jnp.int32).reshape(_NS, _EPT)
    dst = edge_index[1].astype(jnp.int32).reshape(_NS, _EPT)
    src = jnp.concatenate(
        [src, jnp.zeros((_NS, pad), jnp.int32)], axis=1).reshape(-1)
    dst = jnp.concatenate(
        [dst, jnp.full((_NS, pad), _N, jnp.int32)], axis=1).reshape(-1)
    zrow = jnp.zeros((_NACC, _H), jnp.float32)
    zdeg = jnp.zeros((_NACC,), jnp.float32)

    yA1, yB1, z1 = _tc_pre(x, Wl1, Wr1, b1)
    s1, deg1 = _seg_sum_sc(yA1, yB1, src, dst, zrow, zdeg, with_deg=True)
    deg = deg1[:_N].reshape(_N, 1)
    y2A, y2B, z2 = _tc_mid(s1[0], s1[1], deg, z1, Wl2, Wr2, b2)
    (s2,) = _seg_sum_sc(y2A, y2B, src, dst, zrow, zdeg, with_deg=False)
    return _tc_post(s2[0], s2[1], deg, z2)


# speedup vs baseline: 1.7978x; 1.7978x over previous
import functools

import jax
import jax.numpy as jnp
from jax import lax
from jax.experimental import pallas as pl
from jax.experimental.pallas import tpu as pltpu
from jax.experimental.pallas import tpu_sc as plsc

_N = 10000
_E = 160000
_D = 256
_H = _D // 2
_NC = 2
_NS = 16
_EPT = _E // _NS
_C = 80
_NCHUNK = _EPT // _C
_EPTP = _NCHUNK * _C
_NACC = _N + 16
_RPT = 624
_RTAIL = _N - _NS * _RPT
_BN = 1000


def _seg_sum_sc(yA, yB, src, dst, zrow, zdeg, with_deg):
    mesh = plsc.VectorSubcoreMesh(core_axis_name="c", subcore_axis_name="s")
    out_type = [jax.ShapeDtypeStruct((_NC, _N, _H), jnp.float32)]
    if with_deg:
        out_type.append(jax.ShapeDtypeStruct((_NACC,), jnp.float32))

    @functools.partial(
        pl.kernel,
        out_type=tuple(out_type),
        mesh=mesh,
        scratch_types=[
            pltpu.VMEM((_EPTP,), jnp.int32),
            pltpu.VMEM((_EPTP,), jnp.int32),
            pltpu.VMEM((_C,), jnp.int32),
            pltpu.VMEM((_C,), jnp.int32),
            pltpu.VMEM((_C, _H), jnp.float32),
            pltpu.VMEM((_C, _H), jnp.float32),
            pltpu.VMEM((_C,), jnp.float32),
            pltpu.VMEM_SHARED((_NACC, _H), jnp.float32),
            pltpu.VMEM_SHARED((_NACC,), jnp.float32),
            pltpu.SemaphoreType.DMA,
            pltpu.SemaphoreType.DMA,
        ],
    )
    def k(yA_hbm, yB_hbm, src_hbm, dst_hbm, zrow_hbm, zdeg_hbm,
          *out_and_scratch):
        if with_deg:
            s_out, deg_out = out_and_scratch[:2]
            scr = out_and_scratch[2:]
        else:
            s_out = out_and_scratch[0]
            scr = out_and_scratch[1:]
        (src_st, dst_st, dstA, dstB, bufA, bufB, ones_v,
         acc_sh, deg_sh, semA, semB) = scr
        c = lax.axis_index("c")
        s = lax.axis_index("s")
        if with_deg:
            for i in range(_C // 16):
                ones_v[pl.ds(i * 16, 16)] = jnp.ones((16,), jnp.float32)

        base = s * _EPTP
        pltpu.async_copy(src_hbm.at[pl.ds(base, _EPTP)], src_st, semA)
        pltpu.async_copy(dst_hbm.at[pl.ds(base, _EPTP)], dst_st, semB)

        @pl.when(s == 0)
        def _():
            pltpu.sync_copy(zrow_hbm, acc_sh)
            if with_deg:
                @pl.when(c == 0)
                def _():
                    pltpu.sync_copy(zdeg_hbm, deg_sh)

        pltpu.make_async_copy(src_hbm.at[pl.ds(0, _EPTP)], src_st, semA).wait()
        pltpu.make_async_copy(dst_hbm.at[pl.ds(0, _EPTP)], dst_st, semB).wait()
        plsc.subcore_barrier()

        def gather(kk, buf, sem):
            idx = src_st.at[pl.ds(kk * _C, _C)]

            @pl.when(c == 0)
            def _():
                pltpu.async_copy(yA_hbm.at[idx], buf, sem)

            @pl.when(c == 1)
            def _():
                pltpu.async_copy(yB_hbm.at[idx], buf, sem)

        def wait(buf, sem):
            pltpu.make_async_copy(
                yA_hbm.at[src_st.at[pl.ds(0, _C)]], buf, sem).wait()

        def load_dst(kk, dst_v):
            for i in range(_C // 16):
                dst_v[pl.ds(i * 16, 16)] = dst_st[pl.ds(kk * _C + i * 16, 16)]

        def scatter(kk, buf, dst_v):
            pltpu.sync_copy(buf, acc_sh.at[dst_v], add=True)
            if with_deg:
                @pl.when(c == 0)
                def _():
                    pltpu.sync_copy(ones_v, deg_sh.at[dst_v], add=True)

        gather(0, bufA, semA)

        @pl.loop(0, (_NCHUNK - 1) // 2)
        def _(i):
            k0 = 2 * i
            gather(k0 + 1, bufB, semB)
            load_dst(k0, dstA)
            wait(bufA, semA)
            scatter(k0, bufA, dstA)
            gather(k0 + 2, bufA, semA)
            load_dst(k0 + 1, dstB)
            wait(bufB, semB)
            scatter(k0 + 1, bufB, dstB)

        load_dst(_NCHUNK - 1, dstA)
        wait(bufA, semA)
        scatter(_NCHUNK - 1, bufA, dstA)

        plsc.subcore_barrier()
        pltpu.sync_copy(acc_sh.at[pl.ds(s * _RPT, _RPT)],
                        s_out.at[c, pl.ds(s * _RPT, _RPT)])

        @pl.when(s == _NS - 1)
        def _():
            pltpu.sync_copy(acc_sh.at[pl.ds(_NS * _RPT, _RTAIL)],
                            s_out.at[c, pl.ds(_NS * _RPT, _RTAIL)])

        if with_deg:
            @pl.when(jnp.logical_and(s == 0, c == 0))
            def _():
                pltpu.sync_copy(deg_sh, deg_out)

    return k(yA, yB, src, dst, zrow, zdeg)


def _mm(a, b):
    return jnp.dot(a, b, preferred_element_type=jnp.float32)


def _tc_pre(x, Wl, Wr, b):
    def body(x_ref, wl_ref, wr_ref, b_ref, yA_ref, yB_ref, z_ref):
        xb = x_ref[...]
        y = _mm(xb, wl_ref[...])
        yA_ref[...] = y[:, :_H]
        yB_ref[...] = y[:, _H:]
        z_ref[...] = _mm(xb, wr_ref[...]) + b_ref[...]

    return pl.pallas_call(
        body,
        grid=(_N // _BN,),
        in_specs=[
            pl.BlockSpec((_BN, _D), lambda i: (i, 0)),
            pl.BlockSpec((_D, _D), lambda i: (0, 0)),
            pl.BlockSpec((_D, _D), lambda i: (0, 0)),
            pl.BlockSpec((1, _D), lambda i: (0, 0)),
        ],
        out_specs=[
            pl.BlockSpec((_BN, _H), lambda i: (i, 0)),
            pl.BlockSpec((_BN, _H), lambda i: (i, 0)),
            pl.BlockSpec((_BN, _D), lambda i: (i, 0)),
        ],
        out_shape=[
            jax.ShapeDtypeStruct((_N, _H), jnp.float32),
            jax.ShapeDtypeStruct((_N, _H), jnp.float32),
            jax.ShapeDtypeStruct((_N, _D), jnp.float32),
        ],
    )(x, Wl, Wr, b.reshape(1, _D))


def _tc_mid(sA, sB, deg, z, Wl, Wr, b):
    def body(sA_ref, sB_ref, deg_ref, z_ref, wl_ref, wr_ref, b_ref,
             yA_ref, yB_ref, z2_ref):
        rd = 1.0 / jnp.maximum(deg_ref[...], 1.0)
        zb = z_ref[...]
        hA = jnp.maximum(sA_ref[...] * rd + zb[:, :_H], 0.0)
        hB = jnp.maximum(sB_ref[...] * rd + zb[:, _H:], 0.0)
        wl = wl_ref[...]
        wr = wr_ref[...]
        y2 = _mm(hA, wl[:_H, :]) + _mm(hB, wl[_H:, :])
        yA_ref[...] = y2[:, :_H]
        yB_ref[...] = y2[:, _H:]
        z2_ref[...] = _mm(hA, wr[:_H, :]) + _mm(hB, wr[_H:, :]) + b_ref[...]

    return pl.pallas_call(
        body,
        grid=(_N // _BN,),
        in_specs=[
            pl.BlockSpec((_BN, _H), lambda i: (i, 0)),
            pl.BlockSpec((_BN, _H), lambda i: (i, 0)),
            pl.BlockSpec((_BN, 1), lambda i: (i, 0)),
            pl.BlockSpec((_BN, _D), lambda i: (i, 0)),
            pl.BlockSpec((_D, _D), lambda i: (0, 0)),
            pl.BlockSpec((_D, _D), lambda i: (0, 0)),
            pl.BlockSpec((1, _D), lambda i: (0, 0)),
        ],
        out_specs=[
            pl.BlockSpec((_BN, _H), lambda i: (i, 0)),
            pl.BlockSpec((_BN, _H), lambda i: (i, 0)),
            pl.BlockSpec((_BN, _D), lambda i: (i, 0)),
        ],
        out_shape=[
            jax.ShapeDtypeStruct((_N, _H), jnp.float32),
            jax.ShapeDtypeStruct((_N, _H), jnp.float32),
            jax.ShapeDtypeStruct((_N, _D), jnp.float32),
        ],
    )(sA, sB, deg, z, Wl, Wr, b.reshape(1, _D))


def _tc_post(sA, sB, deg, z):
    def body(sA_ref, sB_ref, deg_ref, z_ref, o_ref):
        rd = 1.0 / jnp.maximum(deg_ref[...], 1.0)
        o_ref[...] = jnp.concatenate(
            [sA_ref[...] * rd, sB_ref[...] * rd], axis=1) + z_ref[...]

    return pl.pallas_call(
        body,
        grid=(_N // _BN,),
        in_specs=[
            pl.BlockSpec((_BN, _H), lambda i: (i, 0)),
            pl.BlockSpec((_BN, _H), lambda i: (i, 0)),
            pl.BlockSpec((_BN, 1), lambda i: (i, 0)),
            pl.BlockSpec((_BN, _D), lambda i: (i, 0)),
        ],
        out_specs=pl.BlockSpec((_BN, _D), lambda i: (i, 0)),
        out_shape=jax.ShapeDtypeStruct((_N, _D), jnp.float32),
    )(sA, sB, deg, z)


def kernel(x, edge_index, Wl1, Wr1, b1, Wl2, Wr2, b2):
    pad = _EPTP - _EPT
    src = edge_index[0].astype(jnp.int32).reshape(_NS, _EPT)
    dst = edge_index[1].astype(jnp.int32).reshape(_NS, _EPT)
    src = jnp.concatenate(
        [src, jnp.zeros((_NS, pad), jnp.int32)], axis=1).reshape(-1)
    dst = jnp.concatenate(
        [dst, jnp.full((_NS, pad), _N, jnp.int32)], axis=1).reshape(-1)
    zrow = jnp.zeros((_NACC, _H), jnp.float32)
    zdeg = jnp.zeros((_NACC,), jnp.float32)

    yA1, yB1, z1 = _tc_pre(x, Wl1, Wr1, b1)
    s1, deg1 = _seg_sum_sc(yA1, yB1, src, dst, zrow, zdeg, with_deg=True)
    deg = deg1[:_N].reshape(_N, 1)
    y2A, y2B, z2 = _tc_mid(s1[0], s1[1], deg, z1, Wl2, Wr2, b2)
    (s2,) = _seg_sum_sc(y2A, y2B, src, dst, zrow, zdeg, with_deg=False)
    return _tc_post(s2[0], s2[1], deg, z2)
